# baseline (speedup 1.0000x reference)
import functools
import jax
import jax.numpy as jnp
from jax import lax
from jax.experimental import pallas as pl
from jax.experimental.pallas import tpu as pltpu
from jax.experimental.pallas import tpu_sc as plsc

_B, _I, _D, _C = 2, 16, 128, 13
_HH = 32
_HO = 128
_P = _HH * _HH
_R = _HO * _HO
_L = 16


def _shifted(Xm, xpos, dy, dx):
    o = dy * _HH + dx
    n = Xm.shape[1]
    if o > 0:
        Xs = jnp.concatenate([Xm[o:, :], jnp.zeros((o, n), jnp.float32)], axis=0)
    elif o < 0:
        Xs = jnp.concatenate([jnp.zeros((-o, n), jnp.float32), Xm[:o, :]], axis=0)
    else:
        Xs = Xm
    if dx == -1:
        Xs = jnp.where(xpos > 0, Xs, 0.0)
    elif dx == 1:
        Xs = jnp.where(xpos < _HH - 1, Xs, 0.0)
    return Xs


def _conv_mm(Xm, xpos, w_ref, b, n_out):
    acc = jnp.zeros((_P, n_out), jnp.float32)
    t = 0
    for dy in (-1, 0, 1):
        for dx in (-1, 0, 1):
            Xs = _shifted(Xm, xpos, dy, dx)
            acc = acc + jnp.dot(Xs, w_ref[t * _D:(t + 1) * _D, :],
                                preferred_element_type=jnp.float32)
            t += 1
    return acc + b[None, :]


def _stage1_body(x_ref, w1_ref, b1_ref, w2_ref, b2_ref, wca2_ref, bca2_ref,
                 fcw_ref, fcb_ref, a_ref, at_ref, up_ref, ss_ref):
    X = x_ref[0]
    xpos = lax.broadcasted_iota(jnp.int32, (_P, 1), 0) % _HH
    h = jnp.maximum(_conv_mm(X, xpos, w1_ref[0], b1_ref[0, 0], 2 * _D), 0.0)
    sc1 = h[:, :_D]
    ca1 = h[:, _D:]
    sc2 = jnp.maximum(_conv_mm(sc1, xpos, w2_ref[0], b2_ref[0, 0], _D), 0.0)
    pooled = jnp.mean(sc2, axis=0)
    ssw = jnp.dot(pooled, fcw_ref[0], preferred_element_type=jnp.float32) \
        + fcb_ref[0, 0]
    ss_ref[0, 0] = jnp.broadcast_to(ssw[None, :], (8, _D))
    w = wca2_ref[0, 0]
    c2 = jnp.zeros((_P,), jnp.float32)
    t = 0
    for dy in (-1, 0, 1):
        for dx in (-1, 0, 1):
            Xs = _shifted(ca1, xpos, dy, dx)
            c2 = c2 + jnp.sum(Xs * w[t * _D:(t + 1) * _D][None, :], axis=1)
            t += 1
    c2 = c2 + bca2_ref[0, 0, 0]
    c2m = c2.reshape(_HH, _HH)
    up = jnp.dot(jnp.dot(a_ref[...], c2m, preferred_element_type=jnp.float32),
                 at_ref[...], preferred_element_type=jnp.float32)
    up_ref[0, 0] = up


def _to_mm(W):
    C, O, Iin = W.shape[0], W.shape[1], W.shape[2]
    return W.transpose(0, 3, 4, 2, 1).reshape(C, 9 * Iin, O)


def _sc_assemble(up2, pair_mat, sp, tp, m_mat):
    mesh = plsc.VectorSubcoreMesh(core_axis_name="c", subcore_axis_name="s")
    info = plsc.get_sparse_core_info()
    nc = info.num_cores

    @functools.partial(
        pl.kernel, mesh=mesh,
        out_type=[
            jax.ShapeDtypeStruct((_B * _I, _R), jnp.float32),
            jax.ShapeDtypeStruct((_B * _I, _R), jnp.float32),
        ],
        scratch_types=[
            pltpu.VMEM((_L,), jnp.int32),
            pltpu.VMEM((1, _R), jnp.float32),
            pltpu.VMEM((1, _D), jnp.float32),
            pltpu.VMEM((1, _D), jnp.float32),
            pltpu.VMEM((_L,), jnp.float32),
            pltpu.VMEM((_R,), jnp.float32),
            pltpu.SemaphoreType.DMA,
        ],
    )
    def k(up_hbm, pair_hbm, sp_hbm, tp_hbm, m_hbm, canon_hbm, depth_hbm,
          idx_v, row_v, s_v, t_v, m_v, dp_v, sem):
        wid = lax.axis_index("s") * nc + lax.axis_index("c")
        pltpu.sync_copy(pair_hbm.at[wid], idx_v)
        pltpu.sync_copy(m_hbm.at[wid], m_v)
        idx1 = idx_v.at[pl.ds(0, 1)]
        pltpu.async_copy(up_hbm.at[idx1], row_v, sem).wait()
        pltpu.async_copy(sp_hbm.at[idx1], s_v, sem).wait()
        pltpu.async_copy(tp_hbm.at[idx1], t_v, sem).wait()
        sv = s_v[0, pl.ds(0, _L)]
        tv = t_v[0, pl.ds(0, _L)]
        mv = m_v[...]
        svm = sv * mv
        tvm = tv * mv

        def body(kk, carry):
            sl = pl.ds(kk * _L, _L)
            c = row_v[0, sl] * mv
            row_v[0, sl] = c
            dp_v[sl] = jnp.maximum(c * svm + tvm, 0.001)
            return carry

        lax.fori_loop(0, _R // _L, body, 0, unroll=8)
        pltpu.sync_copy(row_v.at[0], canon_hbm.at[wid])
        pltpu.sync_copy(dp_v, depth_hbm.at[wid])

    return k(up2, pair_mat, sp, tp, m_mat)


def kernel(depth, context, input_feature_map, bin_num, min_depth, max_depth,
           masks, instances, boxes, labels,
           scale_W1, scale_b1, scale_W2, scale_b2, scale_fc_w, scale_fc_b,
           canon_W1, canon_b1, canon_W2, canon_b2):
    f32 = jnp.float32
    X = input_feature_map.transpose(0, 2, 3, 1).reshape(_B, _P, _D)

    w1cat = jnp.concatenate([_to_mm(scale_W1), _to_mm(canon_W1)], axis=2)
    b1cat = jnp.broadcast_to(
        jnp.concatenate([scale_b1, canon_b1], axis=1)[:, None, :], (_C, 8, 2 * _D))
    w2 = _to_mm(scale_W2)
    b2 = jnp.broadcast_to(scale_b2[:, None, :], (_C, 8, _D))
    wca2 = jnp.broadcast_to(_to_mm(canon_W2)[:, None, :, 0], (_C, 8, 9 * _D))
    bca2 = jnp.broadcast_to(canon_b2[:, :, None], (_C, 8, _D))
    fcw = jnp.pad(scale_fc_w, ((0, 0), (0, 0), (0, _D - 2)))
    fcb = jnp.broadcast_to(
        jnp.pad(scale_fc_b, ((0, 0), (0, _D - 2)))[:, None, :], (_C, 8, _D))
    A = jax.image.resize(jnp.eye(_HH, dtype=f32), (_HO, _HH), 'bilinear')
    At = jnp.asarray(A.T)

    up_all, ss_all = pl.pallas_call(
        _stage1_body,
        grid=(_C, _B),
        in_specs=[
            pl.BlockSpec((1, _P, _D), lambda c, s: (s, 0, 0)),
            pl.BlockSpec((1, 9 * _D, 2 * _D), lambda c, s: (c, 0, 0)),
            pl.BlockSpec((1, 8, 2 * _D), lambda c, s: (c, 0, 0)),
            pl.BlockSpec((1, 9 * _D, _D), lambda c, s: (c, 0, 0)),
            pl.BlockSpec((1, 8, _D), lambda c, s: (c, 0, 0)),
            pl.BlockSpec((1, 8, 9 * _D), lambda c, s: (c, 0, 0)),
            pl.BlockSpec((1, 8, _D), lambda c, s: (c, 0, 0)),
            pl.BlockSpec((1, _D, _D), lambda c, s: (c, 0, 0)),
            pl.BlockSpec((1, 8, _D), lambda c, s: (c, 0, 0)),
            pl.BlockSpec((_HO, _HH), lambda c, s: (0, 0)),
            pl.BlockSpec((_HH, _HO), lambda c, s: (0, 0)),
        ],
        out_specs=[
            pl.BlockSpec((1, 1, _HO, _HO), lambda c, s: (c, s, 0, 0)),
            pl.BlockSpec((1, 1, 8, _D), lambda c, s: (c, s, 0, 0)),
        ],
        out_shape=[
            jax.ShapeDtypeStruct((_C, _B, _HO, _HO), f32),
            jax.ShapeDtypeStruct((_C, _B, 8, _D), f32),
        ],
    )(X, w1cat, b1cat, w2, b2, wca2, bca2, fcw, fcb, A, At)

    labf = labels.reshape(_B * _I).astype(jnp.int32)
    b_of = (jnp.arange(_B * _I, dtype=jnp.int32) // _I)
    pair = jnp.clip(labf - 1, 0, _C - 1) * _B + b_of
    pair_mat = jnp.broadcast_to(pair[:, None], (_B * _I, _L))
    m_mat = jnp.broadcast_to(
        (labf > 0).astype(f32)[:, None], (_B * _I, _L))

    up2 = up_all.reshape(_C * _B, _R)
    ss2 = ss_all.reshape(_C * _B, 8 * _D)
    sp = jnp.broadcast_to(ss2[:, 0:1], (_C * _B, _D))
    tp = jnp.broadcast_to(ss2[:, 1:2], (_C * _B, _D))

    canon, dep = _sc_assemble(up2, pair_mat, sp, tp, m_mat)

    lab2 = labels.astype(jnp.int32)
    li = jnp.clip(lab2 - 1, 0, _C - 1)
    s_bt = ss_all[:, :, 0, 0].transpose(1, 0)
    t_bt = ss_all[:, :, 0, 1].transpose(1, 0)
    s_out = jnp.where(lab2 > 0, jnp.take_along_axis(s_bt, li, axis=1), 0.0)
    t_out = jnp.where(lab2 > 0, jnp.take_along_axis(t_bt, li, axis=1), 0.0)

    return (dep.reshape(_B, _I, _HO, _HO),
            canon.reshape(_B, _I, _HO, _HO),
            s_out, t_out)


# speedup vs baseline: 1.1133x; 1.1133x over previous
import functools
import jax
import jax.numpy as jnp
from jax import lax
from jax.experimental import pallas as pl
from jax.experimental.pallas import tpu as pltpu
from jax.experimental.pallas import tpu_sc as plsc

_B, _I, _D, _C = 2, 16, 128, 13
_HH = 32
_HO = 128
_P = _HH * _HH
_K9 = 9 * _D
_L = 16


def _shifted(Xm, xpos, dy, dx):
    o = dy * _HH + dx
    n = Xm.shape[1]
    if o > 0:
        Xs = jnp.concatenate([Xm[o:, :], jnp.zeros((o, n), jnp.float32)], axis=0)
    elif o < 0:
        Xs = jnp.concatenate([jnp.zeros((-o, n), jnp.float32), Xm[:o, :]], axis=0)
    else:
        Xs = Xm
    if dx == -1:
        Xs = jnp.where(xpos > 0, Xs, 0.0)
    elif dx == 1:
        Xs = jnp.where(xpos < _HH - 1, Xs, 0.0)
    return Xs


def _conv_mm(Xm, xpos, w_ref, b, n_out):
    acc = jnp.zeros((_P, n_out), jnp.float32)
    t = 0
    for dy in (-1, 0, 1):
        for dx in (-1, 0, 1):
            Xs = _shifted(Xm, xpos, dy, dx)
            acc = acc + jnp.dot(Xs, w_ref[t * _D:(t + 1) * _D, :],
                                preferred_element_type=jnp.float32)
            t += 1
    return acc + b[None, :]


def _stage1_body(act_ref, x_ref, w1_ref, b1_ref, w2_ref, b2_ref, wca2_ref,
                 bca2_ref, fcw_ref, fcb_ref, c32_ref, ss_ref):
    c = pl.program_id(0)
    s = pl.program_id(1)

    @pl.when(act_ref[c * _B + s] > 0)
    def _():
        X = x_ref[0]
        xpos = lax.broadcasted_iota(jnp.int32, (_P, 1), 0) % _HH
        h = jnp.maximum(_conv_mm(X, xpos, w1_ref[0], b1_ref[0, 0], 2 * _D), 0.0)
        sc1 = h[:, :_D]
        ca1 = h[:, _D:]
        sc2 = jnp.maximum(_conv_mm(sc1, xpos, w2_ref[0], b2_ref[0, 0], _D), 0.0)
        pooled = jnp.mean(sc2, axis=0)
        ssw = jnp.dot(pooled, fcw_ref[0], preferred_element_type=jnp.float32) \
            + fcb_ref[0, 0]
        ss_ref[0, 0] = jnp.broadcast_to(ssw[None, :], (8, _D))
        w = wca2_ref[0, 0]
        c2 = jnp.zeros((_P,), jnp.float32)
        t = 0
        for dy in (-1, 0, 1):
            for dx in (-1, 0, 1):
                Xs = _shifted(ca1, xpos, dy, dx)
                c2 = c2 + jnp.sum(Xs * w[t * _D:(t + 1) * _D][None, :], axis=1)
                t += 1
        c2 = c2 + bca2_ref[0, 0, 0]
        c32_ref[0, 0] = c2.reshape(_HH, _HH)


def _to_mm(W):
    C, O, Iin = W.shape[0], W.shape[1], W.shape[2]
    return W.transpose(0, 3, 4, 2, 1).reshape(C, 9 * Iin, O)


def _stage1(act, X, w1cat, b1cat, w2, b2, wca2, bca2, fcw, fcb):
    f32 = jnp.float32
    return pl.pallas_call(
        _stage1_body,
        grid_spec=pltpu.PrefetchScalarGridSpec(
            num_scalar_prefetch=1,
            grid=(_C, _B),
            in_specs=[
                pl.BlockSpec((1, _P, _D), lambda c, s, a: (s, 0, 0)),
                pl.BlockSpec((1, _K9, 2 * _D), lambda c, s, a: (c, 0, 0)),
                pl.BlockSpec((1, 8, 2 * _D), lambda c, s, a: (c, 0, 0)),
                pl.BlockSpec((1, _K9, _D), lambda c, s, a: (c, 0, 0)),
                pl.BlockSpec((1, 8, _D), lambda c, s, a: (c, 0, 0)),
                pl.BlockSpec((1, 8, _K9), lambda c, s, a: (c, 0, 0)),
                pl.BlockSpec((1, 8, _D), lambda c, s, a: (c, 0, 0)),
                pl.BlockSpec((1, _D, _D), lambda c, s, a: (c, 0, 0)),
                pl.BlockSpec((1, 8, _D), lambda c, s, a: (c, 0, 0)),
            ],
            out_specs=[
                pl.BlockSpec((1, 1, _HH, _HH), lambda c, s, a: (c, s, 0, 0)),
                pl.BlockSpec((1, 1, 8, _D), lambda c, s, a: (c, s, 0, 0)),
            ],
        ),
        out_shape=[
            jax.ShapeDtypeStruct((_C, _B, _HH, _HH), f32),
            jax.ShapeDtypeStruct((_C, _B, 8, _D), f32),
        ],
    )(act, X, w1cat, b1cat, w2, b2, wca2, bca2, fcw, fcb)


def _sc_route(c32_2, pair_mat, sp, tp, m_mat):
    mesh = plsc.VectorSubcoreMesh(core_axis_name="c", subcore_axis_name="s")
    info = plsc.get_sparse_core_info()
    nc = info.num_cores

    @functools.partial(
        pl.kernel, mesh=mesh,
        out_type=[
            jax.ShapeDtypeStruct((_B * _I, _P), jnp.float32),
            jax.ShapeDtypeStruct((_B * _I, _D), jnp.float32),
        ],
        scratch_types=[
            pltpu.VMEM((_L,), jnp.int32),
            pltpu.VMEM((1, _P), jnp.float32),
            pltpu.VMEM((1, _D), jnp.float32),
            pltpu.VMEM((1, _D), jnp.float32),
            pltpu.VMEM((_L,), jnp.float32),
            pltpu.VMEM((_D,), jnp.float32),
            pltpu.SemaphoreType.DMA,
        ],
    )
    def k(c32_hbm, pair_hbm, sp_hbm, tp_hbm, m_hbm, c32r_hbm, str_hbm,
          idx_v, row_v, s_v, t_v, m_v, st_v, sem):
        wid = lax.axis_index("s") * nc + lax.axis_index("c")
        pltpu.sync_copy(pair_hbm.at[wid], idx_v)
        pltpu.sync_copy(m_hbm.at[wid], m_v)
        idx1 = idx_v.at[pl.ds(0, 1)]
        pltpu.async_copy(c32_hbm.at[idx1], row_v, sem).wait()
        pltpu.async_copy(sp_hbm.at[idx1], s_v, sem).wait()
        pltpu.async_copy(tp_hbm.at[idx1], t_v, sem).wait()
        mv = m_v[...] > 0.0
        zer = jnp.zeros((_L,), jnp.float32)
        st_v[pl.ds(0, _L)] = jnp.where(mv, s_v[0, pl.ds(0, _L)], zer)
        st_v[pl.ds(_L, _L)] = jnp.where(mv, t_v[0, pl.ds(0, _L)], zer)

        def body(kk, carry):
            sl = pl.ds(kk * _L, _L)
            row_v[0, sl] = jnp.where(mv, row_v[0, sl], zer)
            return carry

        lax.fori_loop(0, _P // _L, body, 0, unroll=8)
        pltpu.sync_copy(row_v.at[0], c32r_hbm.at[wid])
        pltpu.sync_copy(st_v, str_hbm.at[wid])

    return k(c32_2, pair_mat, sp, tp, m_mat)


def _stage3_body(c32r_ref, str_ref, a_ref, at_ref, canon_ref, depth_ref):
    j = pl.program_id(0)
    c32m = c32r_ref[0]
    up = jnp.dot(jnp.dot(a_ref[...], c32m, preferred_element_type=jnp.float32),
                 at_ref[...], preferred_element_type=jnp.float32)
    s = str_ref[j, 0]
    t = str_ref[j, _L]
    canon_ref[0] = up
    depth_ref[0] = jnp.maximum(up * s + t, 0.001)


def _stage3(c32r3, strow, A, At):
    f32 = jnp.float32
    return pl.pallas_call(
        _stage3_body,
        grid=(_B * _I,),
        in_specs=[
            pl.BlockSpec((1, _HH, _HH), lambda j: (j, 0, 0)),
            pl.BlockSpec((_B * _I, _D), lambda j: (0, 0)),
            pl.BlockSpec((_HO, _HH), lambda j: (0, 0)),
            pl.BlockSpec((_HH, _HO), lambda j: (0, 0)),
        ],
        out_specs=[
            pl.BlockSpec((1, _HO, _HO), lambda j: (j, 0, 0)),
            pl.BlockSpec((1, _HO, _HO), lambda j: (j, 0, 0)),
        ],
        out_shape=[
            jax.ShapeDtypeStruct((_B * _I, _HO, _HO), f32),
            jax.ShapeDtypeStruct((_B * _I, _HO, _HO), f32),
        ],
    )(c32r3, strow, A, At)


def kernel(depth, context, input_feature_map, bin_num, min_depth, max_depth,
           masks, instances, boxes, labels,
           scale_W1, scale_b1, scale_W2, scale_b2, scale_fc_w, scale_fc_b,
           canon_W1, canon_b1, canon_W2, canon_b2):
    f32 = jnp.float32
    X = input_feature_map.transpose(0, 2, 3, 1).reshape(_B, _P, _D)

    w1cat = jnp.concatenate([_to_mm(scale_W1), _to_mm(canon_W1)], axis=2)
    b1cat = jnp.broadcast_to(
        jnp.concatenate([scale_b1, canon_b1], axis=1)[:, None, :], (_C, 8, 2 * _D))
    w2 = _to_mm(scale_W2)
    b2 = jnp.broadcast_to(scale_b2[:, None, :], (_C, 8, _D))
    wca2 = jnp.broadcast_to(_to_mm(canon_W2)[:, None, :, 0], (_C, 8, _K9))
    bca2 = jnp.broadcast_to(canon_b2[:, :, None], (_C, 8, _D))
    fcw = jnp.pad(scale_fc_w, ((0, 0), (0, 0), (0, _D - 2)))
    fcb = jnp.broadcast_to(
        jnp.pad(scale_fc_b, ((0, 0), (0, _D - 2)))[:, None, :], (_C, 8, _D))
    A = jax.image.resize(jnp.eye(_HH, dtype=f32), (_HO, _HH), 'bilinear')
    At = jnp.asarray(A.T)

    lab_bi = labels.astype(jnp.int32)
    act = (lab_bi[None, :, :]
           == (jnp.arange(_C, dtype=jnp.int32) + 1)[:, None, None])
    act = jnp.any(act, axis=2).reshape(_C * _B).astype(jnp.int32)

    c32_all, ss_all = _stage1(act, X, w1cat, b1cat, w2, b2, wca2, bca2,
                              fcw, fcb)

    labf = labels.reshape(_B * _I).astype(jnp.int32)
    b_of = (jnp.arange(_B * _I, dtype=jnp.int32) // _I)
    pair = jnp.clip(labf - 1, 0, _C - 1) * _B + b_of
    pair_mat = jnp.broadcast_to(pair[:, None], (_B * _I, _L))
    m_mat = jnp.broadcast_to((labf > 0).astype(f32)[:, None], (_B * _I, _L))

    c32_2 = c32_all.reshape(_C * _B, _P)
    ss2 = ss_all.reshape(_C * _B, 8 * _D)
    sp = jnp.broadcast_to(ss2[:, 0:1], (_C * _B, _D))
    tp = jnp.broadcast_to(ss2[:, 1:2], (_C * _B, _D))

    c32r, strow = _sc_route(c32_2, pair_mat, sp, tp, m_mat)
    canon, dep = _stage3(c32r.reshape(_B * _I, _HH, _HH), strow, A, At)

    lab2 = labels.astype(jnp.int32)
    li = jnp.clip(lab2 - 1, 0, _C - 1)
    s_bt = ss_all[:, :, 0, 0].transpose(1, 0)
    t_bt = ss_all[:, :, 0, 1].transpose(1, 0)
    s_out = jnp.where(lab2 > 0, jnp.take_along_axis(s_bt, li, axis=1), 0.0)
    t_out = jnp.where(lab2 > 0, jnp.take_along_axis(t_bt, li, axis=1), 0.0)

    return (dep.reshape(_B, _I, _HO, _HO),
            canon.reshape(_B, _I, _HO, _HO),
            s_out, t_out)
